# baseline (speedup 1.0000x reference)
import functools
import math

import jax
import jax.numpy as jnp
from jax import lax
from jax.experimental import pallas as pl
from jax.experimental.pallas import tpu as pltpu
from jax.experimental.pallas import tpu_sc as plsc

_NC = 2
_NS = 16
_NW = _NC * _NS

_B, _S, _D = 4, 2048, 1024
_SPW = _S // _NW
_CH = 32


def _sc_body(x_hbm, pos_hbm, out_hbm, pos_v, xv):
    c_id = lax.axis_index("c")
    s_id = lax.axis_index("s")
    w = s_id * _NC + c_id
    s0 = w * _SPW
    pltpu.sync_copy(pos_hbm.at[pl.ds(s0 * _D, (_SPW + 2) * _D)], pos_v)
    for b in range(_B):
        for h in range(_SPW // _CH):
            r0 = (b * _S + h * _CH) * _D + s0 * _D
            pltpu.sync_copy(x_hbm.at[pl.ds(r0, _CH * _D)], xv)

            pe_base = (h * _CH + 1) * _D

            def row_body(i, carry):
                xoff = i * _D
                for j in range(_D // 16):
                    o = xoff + j * 16
                    pe = pos_v[pl.ds(pe_base + o, 16)]
                    xv[pl.ds(o, 16)] = xv[pl.ds(o, 16)] * 32.0 + pe
                return carry

            lax.fori_loop(0, _CH, row_body, 0)
            pltpu.sync_copy(xv, out_hbm.at[pl.ds(r0, _CH * _D)])


def kernel(x, pos_table):
    B, S, D = x.shape
    x_flat = x.reshape(-1)
    pos_flat = pos_table.reshape(-1)
    mesh = plsc.VectorSubcoreMesh(core_axis_name="c", subcore_axis_name="s")
    out = pl.kernel(
        _sc_body,
        mesh=mesh,
        out_type=jax.ShapeDtypeStruct((B * S * D,), jnp.float32),
        scratch_types=[
            pltpu.VMEM(((_SPW + 2) * _D,), jnp.float32),
            pltpu.VMEM((_CH * _D,), jnp.float32),
        ],
    )(x_flat, pos_flat)
    return out.reshape(B, S, D)


# speedup vs baseline: 5.5644x; 5.5644x over previous
import math

import jax
import jax.numpy as jnp
from jax.experimental import pallas as pl
from jax.experimental.pallas import tpu as pltpu


_BS = 1024
_CH = 256


def _pe_add_kernel(x_ref, pos_ref, o_ref):
    factor = math.sqrt(x_ref.shape[-1])
    S = pos_ref.shape[0] - 2
    j = pl.program_id(1)
    nj = pl.num_programs(1)
    n_chunks = _BS // _CH
    for c in range(n_chunks):
        base = c * _CH
        is_last_chunk = c == n_chunks - 1

        if is_last_chunk:
            @pl.when(j == nj - 1)
            def _tail():
                start = S - _CH
                win = pos_ref[pl.ds(start, _CH), :]
                last = pos_ref[pl.ds(S, 2), :]
                pe = jnp.concatenate([win[1:_CH, :], last[0:1, :]], axis=0)
                o_ref[0, pl.ds(base, _CH), :] = (
                    x_ref[0, pl.ds(base, _CH), :] * factor + pe
                )

            @pl.when(j != nj - 1)
            def _mid():
                win = pos_ref[pl.ds(j * _BS + base, _CH + 8), :]
                pe = win[1:_CH + 1, :]
                o_ref[0, pl.ds(base, _CH), :] = (
                    x_ref[0, pl.ds(base, _CH), :] * factor + pe
                )
        else:
            win = pos_ref[pl.ds(j * _BS + base, _CH + 8), :]
            pe = win[1:_CH + 1, :]
            o_ref[0, pl.ds(base, _CH), :] = (
                x_ref[0, pl.ds(base, _CH), :] * factor + pe
            )


def kernel(x, pos_table):
    B, S, D = x.shape
    return pl.pallas_call(
        _pe_add_kernel,
        grid=(B, S // _BS),
        in_specs=[
            pl.BlockSpec((1, _BS, D), lambda i, j: (i, j, 0)),
            pl.BlockSpec(pos_table.shape, lambda i, j: (0, 0)),
        ],
        out_specs=pl.BlockSpec((1, _BS, D), lambda i, j: (i, j, 0)),
        out_shape=jax.ShapeDtypeStruct((B, S, D), x.dtype),
        compiler_params=pltpu.CompilerParams(
            dimension_semantics=("parallel", "parallel"),
        ),
    )(x, pos_table)


# speedup vs baseline: 5.8344x; 1.0485x over previous
import math

import jax
import jax.numpy as jnp
from jax.experimental import pallas as pl
from jax.experimental.pallas import tpu as pltpu


_CH = 256


def _pe_add_kernel(x_ref, pos_ref, o_ref):
    factor = math.sqrt(x_ref.shape[-1])
    S = x_ref.shape[1]
    n_chunks = S // _CH
    for c in range(n_chunks):
        base = c * _CH
        if base + _CH < S:
            win = pos_ref[pl.ds(base, _CH + 8), :]
            pe = win[1:_CH + 1, :]
        else:
            win = pos_ref[pl.ds(base, _CH), :]
            last = pos_ref[pl.ds(S, 2), :]
            pe = jnp.concatenate([win[1:_CH, :], last[0:1, :]], axis=0)
        o_ref[0, pl.ds(base, _CH), :] = x_ref[0, pl.ds(base, _CH), :] * factor + pe


def kernel(x, pos_table):
    B, S, D = x.shape
    return pl.pallas_call(
        _pe_add_kernel,
        grid=(B,),
        in_specs=[
            pl.BlockSpec((1, S, D), lambda i: (i, 0, 0)),
            pl.BlockSpec(pos_table.shape, lambda i: (0, 0)),
        ],
        out_specs=pl.BlockSpec((1, S, D), lambda i: (i, 0, 0)),
        out_shape=jax.ShapeDtypeStruct((B, S, D), x.dtype),
        compiler_params=pltpu.CompilerParams(
            dimension_semantics=("parallel",),
        ),
    )(x, pos_table)


# speedup vs baseline: 5.8696x; 1.0060x over previous
import math

import jax
import jax.numpy as jnp
from jax.experimental import pallas as pl
from jax.experimental.pallas import tpu as pltpu


_CH = 256


def _pe_add_kernel(x_ref, pos_ref, o_ref):
    factor = math.sqrt(x_ref.shape[-1])
    S = x_ref.shape[1]
    n_chunks = S // _CH
    for c in range(n_chunks):
        base = c * _CH
        if base + _CH < S:
            win = pos_ref[pl.ds(base, _CH + 8), :]
            pe = win[1:_CH + 1, :]
        else:
            win = pos_ref[pl.ds(base, _CH), :]
            last = pos_ref[pl.ds(S, 2), :]
            pe = jnp.concatenate([win[1:_CH, :], last[0:1, :]], axis=0)
        o_ref[0, pl.ds(base, _CH), :] = x_ref[0, pl.ds(base, _CH), :] * factor + pe


def kernel(x, pos_table):
    B, S, D = x.shape
    return pl.pallas_call(
        _pe_add_kernel,
        grid=(B,),
        in_specs=[
            pl.BlockSpec((1, S, D), lambda i: (i, 0, 0)),
            pl.BlockSpec(pos_table.shape, lambda i: (0, 0)),
        ],
        out_specs=pl.BlockSpec((1, S, D), lambda i: (i, 0, 0)),
        out_shape=jax.ShapeDtypeStruct((B, S, D), x.dtype),
        compiler_params=pltpu.CompilerParams(
            dimension_semantics=("arbitrary",),
        ),
    )(x, pos_table)
